# baseline (speedup 1.0000x reference)
import jax
import jax.numpy as jnp
from jax import lax
from jax.experimental import pallas as pl
from jax.experimental.pallas import tpu as pltpu
from jax.experimental.pallas import tpu_sc as plsc

K = 10
ALPHA = 0.1

NC = 2
NS = 16
LANES = 16

N = 10000
D = 128
E = 320000
NP = 10240
RPT = NP // NS
EB = 128
NBLK = 80
EPT = NBLK * EB
EPAD = NC * NS * EPT
BR = 1024


def _sc_deg_body(rowe, deg_out, degsh, degv, ones, rix):
    c = lax.axis_index("c")
    s = lax.axis_index("s")
    r0 = s * RPT

    for i in range(EB // LANES):
        ones[pl.ds(i * LANES, LANES)] = jnp.ones((LANES,), jnp.float32)

    def fill_zero(i, _):
        degv[pl.ds(i * LANES, LANES)] = jnp.zeros((LANES,), jnp.float32)
        return 0
    lax.fori_loop(0, RPT // LANES, fill_zero, 0)
    pltpu.sync_copy(degv, degsh.at[pl.ds(r0, RPT)])
    plsc.subcore_barrier()

    for h in range(NH):
        pltpu.sync_copy(rowe.at[c, s, h], rix)

        def deg_blk(b, _):
            pltpu.sync_copy(ones, degsh.at[rix.at[b]], add=True)
            return 0
        lax.fori_loop(0, HB, deg_blk, 0)
    plsc.subcore_barrier()

    pltpu.sync_copy(degsh.at[pl.ds(r0, RPT)], degv)
    pltpu.sync_copy(degv, deg_out.at[c, pl.ds(r0, RPT)])


DEPTH = 2
NH = 2
HB = NBLK // NH


def _sc_step_body(g_hbm, rowe, cole, tout,
                  T, gb0, gb1, rix, cix, gs0, gs1, ss0, ss1):
    c = lax.axis_index("c")
    s = lax.axis_index("s")
    r0 = s * RPT
    gbs = (gb0, gb1)
    gsm = (gs0, gs1)
    ssm = (ss0, ss1)

    pltpu.sync_copy(g_hbm.at[pl.ds(r0, RPT)], T.at[pl.ds(r0, RPT)])
    plsc.subcore_barrier()

    for h in range(NH):
        pltpu.sync_copy(rowe.at[c, s, h], rix)
        pltpu.sync_copy(cole.at[c, s, h], cix)
        for b in range(DEPTH):
            pltpu.async_copy(g_hbm.at[cix.at[b]], gbs[b], gsm[b])
        for b in range(HB):
            j = b % DEPTH
            pltpu.make_async_copy(g_hbm.at[cix.at[b]], gbs[j], gsm[j]).wait()
            pltpu.async_copy(gbs[j], T.at[rix.at[b]], ssm[j], add=True)
            if b >= 1:
                k = (b - 1) % DEPTH
                pltpu.make_async_copy(
                    gbs[k], T.at[rix.at[b - 1]], ssm[k]).wait()
                if b + 1 < HB:
                    pltpu.async_copy(g_hbm.at[cix.at[b + 1]], gbs[k], gsm[k])
        pltpu.make_async_copy(
            gbs[(HB - 1) % DEPTH], T.at[rix.at[HB - 1]],
            ssm[(HB - 1) % DEPTH]).wait()
    plsc.subcore_barrier()

    pltpu.sync_copy(T.at[pl.ds(r0, RPT)], tout.at[c, pl.ds(r0, RPT)])


def _tc_init_body(x_ref, degT_ref, g0_ref, s1_ref, s2_ref):
    dg = degT_ref[...]
    deg = dg[:, 0:1] + dg[:, 1:2] + 1.0
    d1 = lax.rsqrt(deg)
    d2 = 1.0 / deg
    g0_ref[...] = x_ref[...] * d1
    s1_ref[...] = jnp.broadcast_to(d1, (BR, D))
    s2_ref[...] = jnp.broadcast_to(d2, (BR, D))


def _tc_step_body(t0_ref, t1_ref, g_ref, s1_ref, s2_ref, acc_ref,
                  gout_ref, accout_ref):
    t = t0_ref[...] + t1_ref[...] - g_ref[...]
    gout_ref[...] = s2_ref[...] * t
    accout_ref[...] = acc_ref[...] + s1_ref[...] * t


def _tc_final_body(acc_ref, x_ref, w_ref, b_ref, o_ref):
    pre = acc_ref[...] * ((1.0 - ALPHA) / K) + ALPHA * x_ref[...]
    o_ref[...] = (
        jnp.dot(pre, w_ref[...], preferred_element_type=jnp.float32)
        + b_ref[...]
    )


def _row_block(i):
    return (i, 0)


def kernel(x, edge_index, W0, b0):
    n, d = x.shape
    e = edge_index.shape[1]
    assert (n, d, e) == (N, D, E)

    xp = jnp.zeros((NP, D), jnp.float32).at[:N].set(x)
    pad = jnp.full((EPAD - E,), NP - 1, jnp.int32)
    rowp = jnp.concatenate([edge_index[0], pad]).reshape(NC, NS, NH, HB, EB)
    colp = jnp.concatenate([edge_index[1], pad]).reshape(NC, NS, NH, HB, EB)

    mesh = plsc.VectorSubcoreMesh(core_axis_name="c", subcore_axis_name="s")

    sc_deg = pl.kernel(
        _sc_deg_body,
        out_type=[pltpu.HBM((NC, NP), jnp.float32)],
        mesh=mesh,
        scratch_types=[
            pltpu.VMEM_SHARED((NP,), jnp.float32),
            pltpu.VMEM((RPT,), jnp.float32),
            pltpu.VMEM((EB,), jnp.float32),
            pltpu.VMEM((HB, EB), jnp.int32),
        ],
        name="ssgc_sc_deg",
    )
    (deg2,) = sc_deg(rowp)

    sc_step = pl.kernel(
        _sc_step_body,
        out_type=[pltpu.HBM((NC, NP, D), jnp.float32)],
        mesh=mesh,
        scratch_types=[
            pltpu.VMEM_SHARED((NP, D), jnp.float32),
            pltpu.VMEM((EB, D), jnp.float32),
            pltpu.VMEM((EB, D), jnp.float32),
            pltpu.VMEM((HB, EB), jnp.int32),
            pltpu.VMEM((HB, EB), jnp.int32),
            pltpu.SemaphoreType.DMA,
            pltpu.SemaphoreType.DMA,
            pltpu.SemaphoreType.DMA,
            pltpu.SemaphoreType.DMA,
        ],
        name="ssgc_sc_step",
    )

    grid = (NP // BR,)
    g0, s1e, s2e = pl.pallas_call(
        _tc_init_body,
        grid=grid,
        in_specs=[
            pl.BlockSpec((BR, D), _row_block),
            pl.BlockSpec((BR, NC), _row_block),
        ],
        out_specs=[
            pl.BlockSpec((BR, D), _row_block),
            pl.BlockSpec((BR, D), _row_block),
            pl.BlockSpec((BR, D), _row_block),
        ],
        out_shape=[
            jax.ShapeDtypeStruct((NP, D), jnp.float32),
            jax.ShapeDtypeStruct((NP, D), jnp.float32),
            jax.ShapeDtypeStruct((NP, D), jnp.float32),
        ],
        name="ssgc_tc_init",
    )(xp, deg2.transpose(1, 0))

    tc_step = pl.pallas_call(
        _tc_step_body,
        grid=grid,
        in_specs=[pl.BlockSpec((BR, D), _row_block)] * 6,
        out_specs=[
            pl.BlockSpec((BR, D), _row_block),
            pl.BlockSpec((BR, D), _row_block),
        ],
        out_shape=[
            jax.ShapeDtypeStruct((NP, D), jnp.float32),
            jax.ShapeDtypeStruct((NP, D), jnp.float32),
        ],
        name="ssgc_tc_step",
    )

    g = g0
    acch = jnp.zeros((NP, D), jnp.float32)
    for _ in range(K):
        (tpart,) = sc_step(g, rowp, colp)
        g, acch = tc_step(tpart[0], tpart[1], g, s1e, s2e, acch)

    out = pl.pallas_call(
        _tc_final_body,
        grid=grid,
        in_specs=[
            pl.BlockSpec((BR, D), _row_block),
            pl.BlockSpec((BR, D), _row_block),
            pl.BlockSpec((D, 64), lambda i: (0, 0)),
            pl.BlockSpec((1, 64), lambda i: (0, 0)),
        ],
        out_specs=pl.BlockSpec((BR, 64), _row_block),
        out_shape=jax.ShapeDtypeStruct((NP, 64), jnp.float32),
        name="ssgc_tc_final",
    )(acch, xp, W0, b0.reshape(1, 64))

    return out[:N]


# speedup vs baseline: 1.0381x; 1.0381x over previous
import jax
import jax.numpy as jnp
from jax import lax
from jax.experimental import pallas as pl
from jax.experimental.pallas import tpu as pltpu
from jax.experimental.pallas import tpu_sc as plsc

K = 10
ALPHA = 0.1

NC = 2
NS = 16
LANES = 16

N = 10000
D = 128
E = 320000
NP = 10240
RPT = NP // NS
EB = 128
NBLK = 80
EPT = NBLK * EB
EPAD = NC * NS * EPT
BR = 1024


def _sc_deg_body(rowe, deg_out, degsh, degv, ones, rix):
    c = lax.axis_index("c")
    s = lax.axis_index("s")
    r0 = s * RPT

    for i in range(EB // LANES):
        ones[pl.ds(i * LANES, LANES)] = jnp.ones((LANES,), jnp.float32)

    def fill_zero(i, _):
        degv[pl.ds(i * LANES, LANES)] = jnp.zeros((LANES,), jnp.float32)
        return 0
    lax.fori_loop(0, RPT // LANES, fill_zero, 0)
    pltpu.sync_copy(degv, degsh.at[pl.ds(r0, RPT)])
    plsc.subcore_barrier()

    for h in range(NH):
        pltpu.sync_copy(rowe.at[c, s, h], rix)

        def deg_blk(b, _):
            pltpu.sync_copy(ones, degsh.at[rix.at[b]], add=True)
            return 0
        lax.fori_loop(0, HB, deg_blk, 0)
    plsc.subcore_barrier()

    pltpu.sync_copy(degsh.at[pl.ds(r0, RPT)], degv)
    pltpu.sync_copy(degv, deg_out.at[c, pl.ds(r0, RPT)])


DEPTH = 2
NH = 2
HB = NBLK // NH


def _sc_step_body(g_hbm, rowe, cole, tout,
                  T, gb0, gb1, rix, cix, gs0, gs1):
    c = lax.axis_index("c")
    s = lax.axis_index("s")
    r0 = s * RPT
    gbs = (gb0, gb1)
    gsm = (gs0, gs1)

    pltpu.sync_copy(g_hbm.at[pl.ds(r0, RPT)], T.at[pl.ds(r0, RPT)])
    plsc.subcore_barrier()

    for h in range(NH):
        pltpu.sync_copy(rowe.at[c, s, h], rix)
        pltpu.sync_copy(cole.at[c, s, h], cix)
        for b in range(DEPTH):
            pltpu.async_copy(g_hbm.at[cix.at[b]], gbs[b], gsm[b])
        for b in range(HB):
            j = b % DEPTH
            pltpu.make_async_copy(g_hbm.at[cix.at[b]], gbs[j], gsm[j]).wait()
            pltpu.sync_copy(gbs[j], T.at[rix.at[b]], add=True)
            if b + DEPTH < HB:
                pltpu.async_copy(g_hbm.at[cix.at[b + DEPTH]], gbs[j], gsm[j])
    plsc.subcore_barrier()

    pltpu.sync_copy(T.at[pl.ds(r0, RPT)], tout.at[c, pl.ds(r0, RPT)])


def _tc_init_body(x_ref, degT_ref, g0_ref, s1_ref, s2_ref):
    dg = degT_ref[...]
    deg = dg[:, 0:1] + dg[:, 1:2] + 1.0
    d1 = lax.rsqrt(deg)
    d2 = 1.0 / deg
    g0_ref[...] = x_ref[...] * d1
    s1_ref[...] = jnp.broadcast_to(d1, (BR, D))
    s2_ref[...] = jnp.broadcast_to(d2, (BR, D))


def _tc_step_body(t0_ref, t1_ref, g_ref, s1_ref, s2_ref, acc_ref,
                  gout_ref, accout_ref):
    t = t0_ref[...] + t1_ref[...] - g_ref[...]
    gout_ref[...] = s2_ref[...] * t
    accout_ref[...] = acc_ref[...] + s1_ref[...] * t


def _tc_final_body(acc_ref, x_ref, w_ref, b_ref, o_ref):
    pre = acc_ref[...] * ((1.0 - ALPHA) / K) + ALPHA * x_ref[...]
    o_ref[...] = (
        jnp.dot(pre, w_ref[...], preferred_element_type=jnp.float32)
        + b_ref[...]
    )


def _row_block(i):
    return (i, 0)


def kernel(x, edge_index, W0, b0):
    n, d = x.shape
    e = edge_index.shape[1]
    assert (n, d, e) == (N, D, E)

    xp = jnp.zeros((NP, D), jnp.float32).at[:N].set(x)
    pad = jnp.full((EPAD - E,), NP - 1, jnp.int32)
    rowp = jnp.concatenate([edge_index[0], pad]).reshape(NC, NS, NH, HB, EB)
    colp = jnp.concatenate([edge_index[1], pad]).reshape(NC, NS, NH, HB, EB)

    mesh = plsc.VectorSubcoreMesh(core_axis_name="c", subcore_axis_name="s")

    sc_deg = pl.kernel(
        _sc_deg_body,
        out_type=[pltpu.HBM((NC, NP), jnp.float32)],
        mesh=mesh,
        scratch_types=[
            pltpu.VMEM_SHARED((NP,), jnp.float32),
            pltpu.VMEM((RPT,), jnp.float32),
            pltpu.VMEM((EB,), jnp.float32),
            pltpu.VMEM((HB, EB), jnp.int32),
        ],
        name="ssgc_sc_deg",
    )
    (deg2,) = sc_deg(rowp)

    sc_step = pl.kernel(
        _sc_step_body,
        out_type=[pltpu.HBM((NC, NP, D), jnp.float32)],
        mesh=mesh,
        scratch_types=[
            pltpu.VMEM_SHARED((NP, D), jnp.float32),
            pltpu.VMEM((EB, D), jnp.float32),
            pltpu.VMEM((EB, D), jnp.float32),
            pltpu.VMEM((HB, EB), jnp.int32),
            pltpu.VMEM((HB, EB), jnp.int32),
            pltpu.SemaphoreType.DMA,
            pltpu.SemaphoreType.DMA,
        ],
        name="ssgc_sc_step",
    )

    grid = (NP // BR,)
    g0, s1e, s2e = pl.pallas_call(
        _tc_init_body,
        grid=grid,
        in_specs=[
            pl.BlockSpec((BR, D), _row_block),
            pl.BlockSpec((BR, NC), _row_block),
        ],
        out_specs=[
            pl.BlockSpec((BR, D), _row_block),
            pl.BlockSpec((BR, D), _row_block),
            pl.BlockSpec((BR, D), _row_block),
        ],
        out_shape=[
            jax.ShapeDtypeStruct((NP, D), jnp.float32),
            jax.ShapeDtypeStruct((NP, D), jnp.float32),
            jax.ShapeDtypeStruct((NP, D), jnp.float32),
        ],
        name="ssgc_tc_init",
    )(xp, deg2.transpose(1, 0))

    tc_step = pl.pallas_call(
        _tc_step_body,
        grid=grid,
        in_specs=[pl.BlockSpec((BR, D), _row_block)] * 6,
        out_specs=[
            pl.BlockSpec((BR, D), _row_block),
            pl.BlockSpec((BR, D), _row_block),
        ],
        out_shape=[
            jax.ShapeDtypeStruct((NP, D), jnp.float32),
            jax.ShapeDtypeStruct((NP, D), jnp.float32),
        ],
        name="ssgc_tc_step",
    )

    g = g0
    acch = jnp.zeros((NP, D), jnp.float32)
    for _ in range(K):
        (tpart,) = sc_step(g, rowp, colp)
        g, acch = tc_step(tpart[0], tpart[1], g, s1e, s2e, acch)

    out = pl.pallas_call(
        _tc_final_body,
        grid=grid,
        in_specs=[
            pl.BlockSpec((BR, D), _row_block),
            pl.BlockSpec((BR, D), _row_block),
            pl.BlockSpec((D, 64), lambda i: (0, 0)),
            pl.BlockSpec((1, 64), lambda i: (0, 0)),
        ],
        out_specs=pl.BlockSpec((BR, 64), _row_block),
        out_shape=jax.ShapeDtypeStruct((NP, 64), jnp.float32),
        name="ssgc_tc_final",
    )(acch, xp, W0, b0.reshape(1, 64))

    return out[:N]
